# baseline (speedup 1.0000x reference)
import jax
import jax.numpy as jnp
from jax.experimental import pallas as pl

N_PEAKS_ = 64
DIM_ = 128


def _cond_body(x_ref, s_ref, b_ref, o_ref):
    s = s_ref[...][:, :, None]
    b = b_ref[...][:, :, None]
    o_ref[...] = x_ref[...] * s + b


def kernel(x, queries, W_scale, W_bias):
    del queries
    rows, dim, t = x.shape
    R = 32
    grid = (rows // R,)
    wblocks = N_PEAKS_ // R

    out = pl.pallas_call(
        _cond_body,
        grid=grid,
        in_specs=[
            pl.BlockSpec((R, dim, t), lambda i: (i, 0, 0)),
            pl.BlockSpec((R, dim), lambda i: (i % wblocks, 0)),
            pl.BlockSpec((R, dim), lambda i: (i % wblocks, 0)),
        ],
        out_specs=pl.BlockSpec((R, dim, t), lambda i: (i, 0, 0)),
        out_shape=jax.ShapeDtypeStruct(x.shape, x.dtype),
    )(x, W_scale, W_bias)
    return out


# speedup vs baseline: 1.0272x; 1.0272x over previous
import jax
import jax.numpy as jnp
from jax.experimental import pallas as pl

N_PEAKS_ = 64
DIM_ = 128


def _cond_body(x_ref, s_ref, b_ref, o_ref):
    s = s_ref[...][:, :, None]
    b = b_ref[...][:, :, None]
    o_ref[...] = x_ref[...] * s + b


def kernel(x, queries, W_scale, W_bias):
    del queries
    rows, dim, t = x.shape
    R = 64
    grid = (rows // R,)
    wblocks = N_PEAKS_ // R

    out = pl.pallas_call(
        _cond_body,
        grid=grid,
        in_specs=[
            pl.BlockSpec((R, dim, t), lambda i: (i, 0, 0)),
            pl.BlockSpec((R, dim), lambda i: (i % wblocks, 0)),
            pl.BlockSpec((R, dim), lambda i: (i % wblocks, 0)),
        ],
        out_specs=pl.BlockSpec((R, dim, t), lambda i: (i, 0, 0)),
        out_shape=jax.ShapeDtypeStruct(x.shape, x.dtype),
    )(x, W_scale, W_bias)
    return out


# speedup vs baseline: 1.0274x; 1.0002x over previous
import jax
import jax.numpy as jnp
from jax.experimental import pallas as pl
from jax.experimental.pallas import tpu as pltpu

N_PEAKS_ = 64
DIM_ = 128


def _cond_body(x_ref, s_ref, b_ref, o_ref):
    s = s_ref[...][:, :, None]
    b = b_ref[...][:, :, None]
    o_ref[...] = x_ref[...] * s + b


def kernel(x, queries, W_scale, W_bias):
    del queries
    rows, dim, t = x.shape
    R = 64
    grid = (rows // R,)
    wblocks = N_PEAKS_ // R

    out = pl.pallas_call(
        _cond_body,
        grid=grid,
        in_specs=[
            pl.BlockSpec((R, dim, t), lambda i: (i, 0, 0)),
            pl.BlockSpec((R, dim), lambda i: (i % wblocks, 0)),
            pl.BlockSpec((R, dim), lambda i: (i % wblocks, 0)),
        ],
        out_specs=pl.BlockSpec((R, dim, t), lambda i: (i, 0, 0)),
        out_shape=jax.ShapeDtypeStruct(x.shape, x.dtype),
        compiler_params=pltpu.CompilerParams(
            dimension_semantics=("parallel",),
        ),
    )(x, W_scale, W_bias)
    return out
